# baseline (speedup 1.0000x reference)
import functools

import jax
import jax.numpy as jnp
from jax.experimental import pallas as pl
from jax.experimental.pallas import tpu as pltpu

_BIN_NUM = (16, 8, 4, 2, 1)


def _tmax_wred_kernel(x_ref, o_ref, acc_ref):
    t = pl.program_id(1)
    blk = jnp.max(x_ref[...].astype(jnp.float32), axis=0)

    @pl.when(t == 0)
    def _init():
        acc_ref[...] = blk

    @pl.when(t > 0)
    def _update():
        acc_ref[...] = jnp.maximum(acc_ref[...], blk)

    @pl.when(t == pl.num_programs(1) - 1)
    def _epilogue():
        tm = acc_ref[...]
        wsum = jnp.sum(tm, axis=0)
        wmax = jnp.max(tm, axis=0)
        o_ref[...] = jnp.stack((wsum, wmax), axis=0)[None].astype(o_ref.dtype)


def _hpp_fc_kernel(m_ref, w_ref, o_ref, *, bin_num, wd):
    m = m_ref[...].astype(jnp.float32)
    wsum = jnp.sum(m[:, 0], axis=0)
    wmax = jnp.max(m[:, 1], axis=0)

    h, n = wsum.shape
    bmax = max(bin_num)
    rows = h // bmax
    s_fine = jnp.sum(wsum.reshape(bmax, rows, n), axis=1)
    m_fine = jnp.max(wmax.reshape(bmax, rows, n), axis=1)

    parts = []
    for b in bin_num:
        g = bmax // b
        if g == 1:
            s_b, m_b = s_fine, m_fine
        else:
            s_b = jnp.sum(s_fine.reshape(b, g, n), axis=1)
            m_b = jnp.max(m_fine.reshape(b, g, n), axis=1)
        parts.append(s_b * (1.0 / (g * rows * wd)) + m_b)
    feat = jnp.concatenate(parts, axis=0)

    o_ref[...] = (feat[:, :, None] * w_ref[...][:, None, :]).astype(o_ref.dtype)


def kernel(sils, fc_w):
    bin_num = _BIN_NUM
    n, s, h, w = sils.shape
    p = sum(bin_num)
    c_out = fc_w.shape[-1]
    bmax = max(bin_num)
    if h % bmax != 0 or any(bmax % b for b in bin_num):
        raise ValueError(f"h={h} must be divisible by the bin pyramid {bin_num}")

    xt = jnp.transpose(sils, (1, 3, 2, 0))
    w2 = fc_w[:, 0, :]

    wp = 2 if w % 2 == 0 else 1
    wb = w // wp
    ts = 1
    for cand in (10, 6, 5, 3, 2):
        if s % cand == 0:
            ts = cand
            break

    partial = pl.pallas_call(
        _tmax_wred_kernel,
        out_shape=jax.ShapeDtypeStruct((wp, 2, h, n), jnp.float32),
        grid=(wp, s // ts),
        in_specs=[pl.BlockSpec((ts, wb, h, n), lambda i, t: (t, i, 0, 0))],
        out_specs=pl.BlockSpec((1, 2, h, n), lambda i, t: (i, 0, 0, 0)),
        scratch_shapes=[pltpu.VMEM((wb, h, n), jnp.float32)],
        compiler_params=pltpu.CompilerParams(
            dimension_semantics=("parallel", "arbitrary"),
            vmem_limit_bytes=100 * 1024 * 1024),
    )(xt)

    cp_ = 2 if c_out % 2 == 0 else 1
    cb = c_out // cp_
    out_pnc = pl.pallas_call(
        functools.partial(_hpp_fc_kernel, bin_num=bin_num, wd=w),
        out_shape=jax.ShapeDtypeStruct((p, n, c_out), sils.dtype),
        grid=(cp_,),
        in_specs=[pl.BlockSpec((wp, 2, h, n), lambda i: (0, 0, 0, 0)),
                  pl.BlockSpec((p, cb), lambda i: (0, i))],
        out_specs=pl.BlockSpec((p, n, cb), lambda i: (0, 0, i)),
        compiler_params=pltpu.CompilerParams(
            dimension_semantics=("parallel",),
            vmem_limit_bytes=100 * 1024 * 1024),
    )(partial, w2)

    return jnp.transpose(out_pnc, (1, 2, 0))
